# Initial kernel scaffold; baseline (speedup 1.0000x reference)
import jax
import jax.numpy as jnp
from jax.experimental import pallas as pl


def kernel(atomic_numbers, weight):
    raise NotImplementedError("write your pallas kernel here")


# speedup vs baseline: 4.3161x; 4.3161x over previous
import functools

import jax
import jax.numpy as jnp
from jax import lax
from jax.experimental import pallas as pl
from jax.experimental.pallas import tpu as pltpu
from jax.experimental.pallas import tpu_sc as plsc

_NC, _NS = 2, 16
_NW = _NC * _NS
_L = 16


@functools.partial(jax.jit, static_argnums=(2, 3))
def _sc_lookup(idx, table_flat, B, D):
    C = 2048
    per_w = B // _NW
    n_chunks = per_w // C
    groups = C // _L
    T = table_flat.shape[0]
    mesh = plsc.VectorSubcoreMesh(
        core_axis_name="c", subcore_axis_name="s",
        num_cores=_NC, num_subcores=_NS,
    )

    @functools.partial(
        pl.kernel,
        out_type=jax.ShapeDtypeStruct((B * D,), jnp.float32),
        mesh=mesh,
        scratch_types=[
            pltpu.VMEM((T,), jnp.float32),
            pltpu.VMEM((C,), jnp.int32),
            pltpu.VMEM((C * D,), jnp.float32),
        ],
        compiler_params=pltpu.CompilerParams(needs_layout_passes=False),
    )
    def k(idx_hbm, tbl_hbm, out_hbm, tbl_v, idx_v, out_v):
        wid = lax.axis_index("s") * _NC + lax.axis_index("c")
        base = wid * per_w
        pltpu.sync_copy(tbl_hbm, tbl_v)
        lane = lax.iota(jnp.int32, 16)
        laneD = lane * D

        def chunk(i, carry):
            off = base + i * C
            pltpu.sync_copy(idx_hbm.at[pl.ds(off, C)], idx_v)

            def group(g, carry2):
                idx16 = idx_v[pl.ds(g * _L, _L)]
                pos = idx16 * D
                dst = laneD + g * (_L * D)
                for d in range(D):
                    val = plsc.load_gather(tbl_v, [pos + d])
                    plsc.store_scatter(out_v, [dst + d], val)
                return carry2

            lax.fori_loop(0, groups, group, 0, unroll=4)
            pltpu.sync_copy(out_v, out_hbm.at[pl.ds(off * D, C * D)])
            return carry

        lax.fori_loop(0, n_chunks, chunk, 0)

    return k(idx, table_flat)


def kernel(atomic_numbers, weight):
    B = atomic_numbers.size
    D = weight.shape[1]
    idx = atomic_numbers.reshape(B).astype(jnp.int32)
    out = _sc_lookup(idx, weight.reshape(-1), B, D)
    return out.reshape(*atomic_numbers.shape, D)


# speedup vs baseline: 4.7023x; 1.0895x over previous
import functools

import jax
import jax.numpy as jnp
from jax import lax
from jax.experimental import pallas as pl
from jax.experimental.pallas import tpu as pltpu
from jax.experimental.pallas import tpu_sc as plsc

_NC, _NS = 2, 16
_NW = _NC * _NS
_L = 16


@functools.partial(jax.jit, static_argnums=(2, 3))
def _sc_lookup(idx, table_flat, B, D):
    C = 2048
    per_w = B // _NW
    n_chunks = per_w // C
    groups = C // _L
    T = table_flat.shape[0]
    mesh = plsc.VectorSubcoreMesh(
        core_axis_name="c", subcore_axis_name="s",
        num_cores=_NC, num_subcores=_NS,
    )

    @functools.partial(
        pl.kernel,
        out_type=jax.ShapeDtypeStruct((B * D,), jnp.float32),
        mesh=mesh,
        scratch_types=[
            pltpu.VMEM((T,), jnp.float32),
            pltpu.VMEM((C,), jnp.int32),
            pltpu.VMEM((C * D,), jnp.float32),
        ],
        compiler_params=pltpu.CompilerParams(needs_layout_passes=False),
    )
    def k(idx_hbm, tbl_hbm, out_hbm, tbl_v, idx_v, out_v):
        wid = lax.axis_index("s") * _NC + lax.axis_index("c")
        base = wid * per_w
        pltpu.sync_copy(tbl_hbm, tbl_v)
        lane = lax.iota(jnp.int32, 16)
        laneD = lane * D

        def chunk(i, carry):
            off = base + i * C
            pltpu.sync_copy(idx_hbm.at[pl.ds(off, C)], idx_v)

            @plsc.parallel_loop(0, groups, unroll=8)
            def group(g):
                idx16 = idx_v[pl.ds(g * _L, _L)]
                pos = idx16 * D
                dst = laneD + g * (_L * D)
                for d in range(D):
                    val = plsc.load_gather(tbl_v, [pos + d])
                    plsc.store_scatter(out_v, [dst + d], val)
            pltpu.sync_copy(out_v, out_hbm.at[pl.ds(off * D, C * D)])
            return carry

        lax.fori_loop(0, n_chunks, chunk, 0)

    return k(idx, table_flat)


def kernel(atomic_numbers, weight):
    B = atomic_numbers.size
    D = weight.shape[1]
    idx = atomic_numbers.reshape(B).astype(jnp.int32)
    out = _sc_lookup(idx, weight.reshape(-1), B, D)
    return out.reshape(*atomic_numbers.shape, D)


# speedup vs baseline: 5.5136x; 1.1725x over previous
import functools

import jax
import jax.numpy as jnp
from jax import lax
from jax.experimental import pallas as pl
from jax.experimental.pallas import tpu as pltpu
from jax.experimental.pallas import tpu_sc as plsc

_NC, _NS = 2, 16
_NW = _NC * _NS
_L = 16


@functools.partial(jax.jit, static_argnums=(2, 3, 4))
def _sc_lookup(an, table, N, M, D):
    R = 4
    per_w = N // _NW
    n_chunks = per_w // R
    groups = (R * M) // _L
    mesh = plsc.VectorSubcoreMesh(
        core_axis_name="c", subcore_axis_name="s",
        num_cores=_NC, num_subcores=_NS,
    )

    @functools.partial(
        pl.kernel,
        out_type=jax.ShapeDtypeStruct((N, M, D), jnp.float32),
        mesh=mesh,
        scratch_types=[
            pltpu.VMEM(table.shape, jnp.float32),
            pltpu.VMEM((R, M), jnp.int32),
            pltpu.VMEM((R, M, D), jnp.float32),
        ],
        compiler_params=pltpu.CompilerParams(needs_layout_passes=False),
    )
    def k(an_hbm, tbl_hbm, out_hbm, tbl_v, idx_v, out_v):
        wid = lax.axis_index("s") * _NC + lax.axis_index("c")
        base = wid * per_w
        pltpu.sync_copy(tbl_hbm, tbl_v)
        lane = lax.iota(jnp.int32, _L)

        def chunk(i, carry):
            row0 = base + i * R
            pltpu.sync_copy(an_hbm.at[pl.ds(row0, R)], idx_v)

            @plsc.parallel_loop(0, groups, unroll=8)
            def group(g):
                q = g * _L + lane
                r = q // M
                c = q - r * M
                idx16 = plsc.load_gather(idx_v, [r, c])
                for d in range(D):
                    dd = jnp.full((_L,), d, jnp.int32)
                    val = plsc.load_gather(tbl_v, [idx16, dd])
                    plsc.store_scatter(out_v, [r, c, dd], val)

            pltpu.sync_copy(out_v, out_hbm.at[pl.ds(row0, R)])
            return carry

        lax.fori_loop(0, n_chunks, chunk, 0)

    return k(an, table)


def kernel(atomic_numbers, weight):
    N, M = atomic_numbers.shape
    D = weight.shape[1]
    return _sc_lookup(atomic_numbers, weight, N, M, D)


# speedup vs baseline: 8.9693x; 1.6268x over previous
import functools

import jax
import jax.numpy as jnp
from jax import lax
from jax.experimental import pallas as pl
from jax.experimental.pallas import tpu as pltpu
from jax.experimental.pallas import tpu_sc as plsc

_NC, _NS = 2, 16
_NW = _NC * _NS
_L = 16
_NBUF = 2


@functools.partial(jax.jit, static_argnums=(2, 3, 4))
def _sc_lookup(an, table_flat, N, M, D):
    R = 2
    per_w = N // _NW
    n_chunks = per_w // R
    groups = (R * M) // _L
    T = table_flat.shape[0]
    mesh = plsc.VectorSubcoreMesh(
        core_axis_name="c", subcore_axis_name="s",
        num_cores=_NC, num_subcores=_NS,
    )

    @functools.partial(
        pl.kernel,
        out_type=jax.ShapeDtypeStruct((N, M, D), jnp.float32),
        mesh=mesh,
        scratch_types=[
            pltpu.VMEM((T,), jnp.float32),
            pltpu.VMEM((_NBUF, R, M), jnp.int32),
            pltpu.VMEM((_NBUF, R, M, D), jnp.float32),
            pltpu.SemaphoreType.DMA((_NBUF,)),
            pltpu.SemaphoreType.DMA((_NBUF,)),
            pltpu.SemaphoreType.DMA,
        ],
        compiler_params=pltpu.CompilerParams(needs_layout_passes=False),
    )
    def k(an_hbm, tbl_hbm, out_hbm, tbl_v, idx_v, out_v, idx_sem, out_sem, tbl_sem):
        wid = lax.axis_index("s") * _NC + lax.axis_index("c")
        base = wid * per_w
        pltpu.async_copy(tbl_hbm, tbl_v, tbl_sem).wait()
        lane = lax.iota(jnp.int32, _L)

        def idx_copy(g, b):
            return pltpu.make_async_copy(
                an_hbm.at[pl.ds(base + g * R, R)], idx_v.at[b], idx_sem.at[b])

        def out_copy(g, b):
            return pltpu.make_async_copy(
                out_v.at[b], out_hbm.at[pl.ds(base + g * R, R)], out_sem.at[b])

        for b in range(_NBUF):
            idx_copy(b, b).start()

        def step(i, carry):
            for b in range(_NBUF):
                g = i * _NBUF + b
                idx_copy(g, b).wait()

                @pl.when(g >= _NBUF)
                def _():
                    out_copy(g - _NBUF, b).wait()

                @plsc.parallel_loop(0, groups, unroll=5)
                def group(g2):
                    q = g2 * _L + lane
                    r = q // M
                    c = q - r * M
                    idx16 = plsc.load_gather(idx_v.at[b], [r, c])
                    pos = idx16 * D
                    for d in range(D):
                        dd = jnp.full((_L,), d, jnp.int32)
                        val = plsc.load_gather(tbl_v, [pos + d])
                        plsc.store_scatter(out_v.at[b], [r, c, dd], val)

                out_copy(g, b).start()

                @pl.when(g + _NBUF < n_chunks)
                def _():
                    idx_copy(g + _NBUF, b).start()
            return carry

        lax.fori_loop(0, n_chunks // _NBUF, step, 0)
        for b in range(_NBUF):
            out_copy(n_chunks - _NBUF + b, b).wait()

    return k(an, table_flat)


def kernel(atomic_numbers, weight):
    N, M = atomic_numbers.shape
    D = weight.shape[1]
    return _sc_lookup(atomic_numbers, weight.reshape(-1), N, M, D)
